# baseline (speedup 1.0000x reference)
import functools

import jax
import jax.numpy as jnp
from jax import lax
from jax.experimental import pallas as pl
from jax.experimental.pallas import tpu as pltpu
from jax.experimental.pallas import tpu_sc as plsc

NEG = -1e30


def _gate_body(x_ref, W1_ref, b1_ref, W2_ref, b2_ref, g_ref, *, P, DH):
    xb = x_ref[...]
    h = jax.lax.dot_general(xb, W1_ref[...], (((1,), (0,)), ((), ())),
                            preferred_element_type=jnp.float32)
    h = jnp.maximum(h + b1_ref[...], 0.0)
    g2 = jax.lax.dot_general(h, W2_ref[...], (((1,), (0,)), ((), ())),
                             preferred_element_type=jnp.float32)
    g_ref[...] = g2 + b2_ref[...]


def _take(v, idx):
    return v.at[idx].get(mode="promise_in_bounds")


def _sc_body(g_hbm, b_hbm, w_hbm, den_hbm, part_hbm,
             bbuf, gbuf, wbuf, Lmax, Lden, pbuf, *, CH, N2, NV):
    c = lax.axis_index("c")
    s = lax.axis_index("s")
    lo = s * CH
    pltpu.sync_copy(b_hbm.at[pl.ds(lo, CH)], bbuf)
    pltpu.sync_copy(g_hbm.at[pl.ds(c * N2 + lo, CH)], gbuf)

    ii = lax.iota(jnp.int32, 16)
    for j in range(8):
        Lmax[pl.ds(j * 16, 16)] = jnp.full((16,), NEG, jnp.float32)

    def phase1(i, carry):
        b = bbuf[pl.ds(i * 16, 16)]
        g = gbuf[pl.ds(i * 16, 16)]
        for d in (1, 2, 4, 8):
            idx = jnp.maximum(ii - d, 0)
            bs = _take(b, idx)
            gs = _take(g, idx)
            g = jnp.where(bs == b, jnp.maximum(g, gs), g)
        bn = _take(b, jnp.minimum(ii + 1, 15))
        last = (b != bn) | (ii == 15)
        cur = plsc.load_gather(Lmax, [b])
        plsc.store_scatter(Lmax, [b], jnp.maximum(cur, g), mask=last)
        return carry

    lax.fori_loop(0, NV, phase1, 0)

    pltpu.sync_copy(Lmax, part_hbm.at[pl.ds(((c * 2 + 0) * 16 + s) * 128, 128)])
    plsc.subcore_barrier()
    pltpu.sync_copy(part_hbm.at[pl.ds((c * 2 + 0) * 16 * 128, 2048)], pbuf)
    for j8 in range(8):
        acc = jnp.full((16,), NEG, jnp.float32)
        for j in range(16):
            acc = jnp.maximum(acc, pbuf[pl.ds(j * 128 + j8 * 16, 16)])
        Lmax[pl.ds(j8 * 16, 16)] = acc

    for j in range(8):
        Lden[pl.ds(j * 16, 16)] = jnp.zeros((16,), jnp.float32)

    def phase2(i, carry):
        b = bbuf[pl.ds(i * 16, 16)]
        g = gbuf[pl.ds(i * 16, 16)]
        gm = plsc.load_gather(Lmax, [b])
        w = jnp.exp(g - gm)
        wbuf[pl.ds(i * 16, 16)] = w
        sv = w
        for d in (1, 2, 4, 8):
            idx = jnp.maximum(ii - d, 0)
            bs = _take(b, idx)
            ss = _take(sv, idx)
            sv = jnp.where((ii >= d) & (bs == b), sv + ss, sv)
        bn = _take(b, jnp.minimum(ii + 1, 15))
        last = (b != bn) | (ii == 15)
        cur = plsc.load_gather(Lden, [b])
        plsc.store_scatter(Lden, [b], cur + sv, mask=last)
        return carry

    lax.fori_loop(0, NV, phase2, 0)

    pltpu.sync_copy(wbuf, w_hbm.at[pl.ds(c * N2 + lo, CH)])
    pltpu.sync_copy(Lden, part_hbm.at[pl.ds(((c * 2 + 1) * 16 + s) * 128, 128)])
    plsc.subcore_barrier()

    @pl.when(s == 0)
    def _reduce_den():
        pltpu.sync_copy(part_hbm.at[pl.ds((c * 2 + 1) * 16 * 128, 2048)], pbuf)
        for j8 in range(8):
            acc = jnp.zeros((16,), jnp.float32)
            for j in range(16):
                acc = acc + pbuf[pl.ds(j * 128 + j8 * 16, 16)]
            Lden[pl.ds(j8 * 16, 16)] = acc
        pltpu.sync_copy(Lden, den_hbm.at[pl.ds(c * 128, 128)])


def _feat_body(x_ref, b_ref, w_ref, W1_ref, b1_ref, fW2_ref, fb2_ref, den_ref,
               out_ref, S_ref, *, R, P, Bn, DH, DE):
    i = pl.program_id(0)

    @pl.when(i == 0)
    def _init():
        S_ref[...] = jnp.zeros((P, Bn, DE), jnp.float32)

    xb = x_ref[...]
    h = jax.lax.dot_general(xb, W1_ref[...], (((1,), (0,)), ((), ())),
                            preferred_element_type=jnp.float32)
    h = jnp.maximum(h + b1_ref[...], 0.0)
    bb = b_ref[...]
    O = bb == jax.lax.broadcasted_iota(jnp.int32, (R, Bn), 1)
    for k in range(P):
        fk = jax.lax.dot_general(h[:, k * DH:(k + 1) * DH], fW2_ref[k],
                                 (((1,), (0,)), ((), ())),
                                 preferred_element_type=jnp.float32)
        fk = fk + fb2_ref[k]
        wk = w_ref[:, k:k + 1]
        E = jnp.where(O, wk, 0.0)
        S_ref[k] = S_ref[k] + jax.lax.dot_general(
            E, fk, (((0,), (0,)), ((), ())), preferred_element_type=jnp.float32)

    @pl.when(i == pl.num_programs(0) - 1)
    def _finish():
        for k in range(P):
            dT = jnp.transpose(den_ref[k:k + 1, :])
            out_ref[k] = jnp.where(dT > 0.0, S_ref[k] / dT, 0.0)


def kernel(x, batch, n_nodes, Omegas, Phis, Lambdas, Omegas_norm, Phis_norm,
           Lambdas_norm, gate_W1, gate_b1, gate_W2, gate_b2, feat_W1, feat_b1,
           feat_W2, feat_b2):
    N, FD = x.shape
    Bn = n_nodes.shape[0]
    P, _, DH = gate_W1.shape
    DE = feat_W2.shape[2]
    R = 2000
    assert N % R == 0
    CH = 6256
    N2 = 16 * CH
    NV = CH // 16

    batch2 = batch.astype(jnp.int32).reshape(N, 1)

    gW1c = jnp.concatenate([gate_W1[k] for k in range(P)], axis=1)
    gb1c = jnp.concatenate([gate_b1[k] for k in range(P)])[None, :]
    gW2c = jnp.zeros((P * DH, P), jnp.float32)
    for k in range(P):
        gW2c = gW2c.at[k * DH:(k + 1) * DH, k].set(gate_W2[k, :, 0])
    gb2c = gate_b2[:, 0][None, :]

    g_np = pl.pallas_call(
        functools.partial(_gate_body, P=P, DH=DH),
        grid=(N // R,),
        in_specs=[
            pl.BlockSpec((R, FD), lambda i: (i, 0)),
            pl.BlockSpec((FD, P * DH), lambda i: (0, 0)),
            pl.BlockSpec((1, P * DH), lambda i: (0, 0)),
            pl.BlockSpec((P * DH, P), lambda i: (0, 0)),
            pl.BlockSpec((1, P), lambda i: (0, 0)),
        ],
        out_specs=pl.BlockSpec((R, P), lambda i: (i, 0)),
        out_shape=jax.ShapeDtypeStruct((N, P), jnp.float32),
    )(x, gW1c, gb1c, gW2c, gb2c)
    gT = g_np.T

    g_pad = jnp.pad(gT, ((0, 0), (0, N2 - N)), constant_values=NEG).reshape(-1)
    b_pad = jnp.pad(batch.astype(jnp.int32), (0, N2 - N), constant_values=Bn)

    mesh = plsc.VectorSubcoreMesh(core_axis_name="c", subcore_axis_name="s")
    sc = pl.kernel(
        functools.partial(_sc_body, CH=CH, N2=N2, NV=NV),
        out_type=(
            jax.ShapeDtypeStruct((P * N2,), jnp.float32),
            jax.ShapeDtypeStruct((P * 128,), jnp.float32),
            jax.ShapeDtypeStruct((P * 2 * 16 * 128,), jnp.float32),
        ),
        mesh=mesh,
        scratch_types=[
            pltpu.VMEM((CH,), jnp.int32),
            pltpu.VMEM((CH,), jnp.float32),
            pltpu.VMEM((CH,), jnp.float32),
            pltpu.VMEM((128,), jnp.float32),
            pltpu.VMEM((128,), jnp.float32),
            pltpu.VMEM((16 * 128,), jnp.float32),
        ],
        compiler_params=pltpu.CompilerParams(needs_layout_passes=False),
    )
    w_flat, den_flat, _ = sc(g_pad, b_pad)
    w2 = w_flat.reshape(P, N2)[:, :N].T
    den = den_flat.reshape(P, 128)[:, :Bn]

    fW1c = jnp.concatenate([feat_W1[k] for k in range(P)], axis=1)
    fb1c = jnp.concatenate([feat_b1[k] for k in range(P)])[None, :]
    fb2r = feat_b2[:, None, :]

    pools = pl.pallas_call(
        functools.partial(_feat_body, R=R, P=P, Bn=Bn, DH=DH, DE=DE),
        grid=(N // R,),
        in_specs=[
            pl.BlockSpec((R, FD), lambda i: (i, 0)),
            pl.BlockSpec((R, 1), lambda i: (i, 0)),
            pl.BlockSpec((R, P), lambda i: (i, 0)),
            pl.BlockSpec((FD, P * DH), lambda i: (0, 0)),
            pl.BlockSpec((1, P * DH), lambda i: (0, 0)),
            pl.BlockSpec((P, DH, DE), lambda i: (0, 0, 0)),
            pl.BlockSpec((P, 1, DE), lambda i: (0, 0, 0)),
            pl.BlockSpec((P, Bn), lambda i: (0, 0)),
        ],
        out_specs=pl.BlockSpec((P, Bn, DE), lambda i: (0, 0, 0)),
        out_shape=jax.ShapeDtypeStruct((P, Bn, DE), jnp.float32),
        scratch_shapes=[pltpu.VMEM((P, Bn, DE), jnp.float32)],
    )(x, batch2, w2, fW1c, fb1c, feat_W2, fb2r, den)

    return jnp.concatenate(
        [pools[k] for k in range(P)]
        + [n_nodes, Omegas, Phis, Lambdas, Omegas_norm, Phis_norm,
           Lambdas_norm], axis=1)


# speedup vs baseline: 1.4573x; 1.4573x over previous
import functools

import jax
import jax.numpy as jnp
from jax.experimental import pallas as pl
from jax.experimental.pallas import tpu as pltpu


def _body(x_ref, b_ref, W1_ref, b1_ref, gW2_ref, gb2_ref, fW2_ref, fb2_ref,
          out_ref, m_ref, d_ref, S_ref, *, R, P, Bn, DH, DE):
    i = pl.program_id(0)

    @pl.when(i == 0)
    def _init():
        m_ref[...] = jnp.full((P, Bn), -1e30, jnp.float32)
        d_ref[...] = jnp.zeros((P, Bn), jnp.float32)
        S_ref[...] = jnp.zeros((P, Bn, DE), jnp.float32)

    xb = x_ref[...].astype(jnp.bfloat16)
    h = jax.lax.dot_general(xb, W1_ref[...], (((1,), (0,)), ((), ())),
                            preferred_element_type=jnp.float32)
    h = jnp.maximum(h.astype(jnp.bfloat16) + b1_ref[...],
                    jnp.bfloat16(0))
    g2 = jax.lax.dot_general(h[:, :P * DH], gW2_ref[...],
                             (((1,), (0,)), ((), ())),
                             preferred_element_type=jnp.float32)
    g2 = g2 + gb2_ref[...]

    bb = b_ref[...]
    seg_ids = jax.lax.broadcasted_iota(jnp.int32, (R, Bn), 1)
    O = bb == seg_ids

    for k in range(P):
        fk = jax.lax.dot_general(h[:, (P + k) * DH:(P + k + 1) * DH],
                                 fW2_ref[k], (((1,), (0,)), ((), ())),
                                 preferred_element_type=jnp.float32)
        fk = fk.astype(jnp.bfloat16) + fb2_ref[k]
        gk = g2[:, k:k + 1]
        masked = jnp.where(O, gk, -1e30)
        bmax = jnp.max(masked, axis=0, keepdims=True)
        m_old = m_ref[k:k + 1, :]
        m_new = jnp.maximum(m_old, bmax)
        scale = jnp.exp(m_old - m_new)
        E = jnp.where(O, jnp.exp(gk - m_new), 0.0)
        d_ref[k:k + 1, :] = (d_ref[k:k + 1, :] * scale
                             + jnp.sum(E, axis=0, keepdims=True))
        S_ref[k] = (S_ref[k] * jnp.transpose(scale)
                    + jax.lax.dot_general(E.astype(jnp.bfloat16), fk,
                                          (((0,), (0,)), ((), ())),
                                          preferred_element_type=jnp.float32))
        m_ref[k:k + 1, :] = m_new

    @pl.when(i == pl.num_programs(0) - 1)
    def _finish():
        for k in range(P):
            dT = jnp.transpose(d_ref[k:k + 1, :])
            out_ref[k] = jnp.where(dT > 0.0, S_ref[k] / dT, 0.0)


def kernel(x, batch, n_nodes, Omegas, Phis, Lambdas, Omegas_norm, Phis_norm,
           Lambdas_norm, gate_W1, gate_b1, gate_W2, gate_b2, feat_W1, feat_b1,
           feat_W2, feat_b2):
    N, FD = x.shape
    Bn = n_nodes.shape[0]
    P, _, DH = gate_W1.shape
    DE = feat_W2.shape[2]
    R = 4000
    assert N % R == 0

    W1all = jnp.concatenate(
        [gate_W1[k] for k in range(P)] + [feat_W1[k] for k in range(P)],
        axis=1).astype(jnp.bfloat16)
    b1all = jnp.concatenate(
        [gate_b1[k] for k in range(P)]
        + [feat_b1[k] for k in range(P)])[None, :].astype(jnp.bfloat16)
    gW2c = jnp.zeros((P * DH, P), jnp.float32)
    for k in range(P):
        gW2c = gW2c.at[k * DH:(k + 1) * DH, k].set(gate_W2[k, :, 0])
    gW2c = gW2c.astype(jnp.bfloat16)
    gb2c = gate_b2[:, 0][None, :]
    fW2b = feat_W2.astype(jnp.bfloat16)
    fb2r = feat_b2[:, None, :].astype(jnp.bfloat16)
    batch2 = batch.astype(jnp.int32).reshape(N, 1)

    body = functools.partial(_body, R=R, P=P, Bn=Bn, DH=DH, DE=DE)
    pools = pl.pallas_call(
        body,
        grid=(N // R,),
        in_specs=[
            pl.BlockSpec((R, FD), lambda i: (i, 0)),
            pl.BlockSpec((R, 1), lambda i: (i, 0)),
            pl.BlockSpec((FD, 2 * P * DH), lambda i: (0, 0)),
            pl.BlockSpec((1, 2 * P * DH), lambda i: (0, 0)),
            pl.BlockSpec((P * DH, P), lambda i: (0, 0)),
            pl.BlockSpec((1, P), lambda i: (0, 0)),
            pl.BlockSpec((P, DH, DE), lambda i: (0, 0, 0)),
            pl.BlockSpec((P, 1, DE), lambda i: (0, 0, 0)),
        ],
        out_specs=pl.BlockSpec((P, Bn, DE), lambda i: (0, 0, 0)),
        out_shape=jax.ShapeDtypeStruct((P, Bn, DE), jnp.float32),
        scratch_shapes=[
            pltpu.VMEM((P, Bn), jnp.float32),
            pltpu.VMEM((P, Bn), jnp.float32),
            pltpu.VMEM((P, Bn, DE), jnp.float32),
        ],
    )(x, batch2, W1all, b1all, gW2c, gb2c, fW2b, fb2r)

    return jnp.concatenate(
        [pools[k] for k in range(P)]
        + [n_nodes, Omegas, Phis, Lambdas, Omegas_norm, Phis_norm,
           Lambdas_norm], axis=1)


# speedup vs baseline: 1.5970x; 1.0959x over previous
import functools

import jax
import jax.numpy as jnp
from jax.experimental import pallas as pl
from jax.experimental.pallas import tpu as pltpu


def _body(x_ref, b_ref, W1_ref, b1_ref, gW2_ref, gb2_ref, fW2_ref, fb2_ref,
          out_ref, m_ref, d_ref, S_ref, *, R, P, Bn, DH, DE):
    i = pl.program_id(0)
    L = P * Bn

    @pl.when(i == 0)
    def _init():
        m_ref[...] = jnp.full((1, L), -1e30, jnp.float32)
        d_ref[...] = jnp.zeros((1, L), jnp.float32)
        S_ref[...] = jnp.zeros((L, P * DE), jnp.float32)

    xb = x_ref[...].astype(jnp.bfloat16)
    h = jax.lax.dot_general(xb, W1_ref[...], (((1,), (0,)), ((), ())),
                            preferred_element_type=jnp.float32)
    h = jnp.maximum(h.astype(jnp.bfloat16) + b1_ref[...],
                    jnp.bfloat16(0))

    gboth = jax.lax.dot_general(h[:, :P * DH], gW2_ref[...],
                                (((1,), (0,)), ((), ())),
                                preferred_element_type=jnp.float32)
    gboth = gboth + gb2_ref[...]

    bb = b_ref[...]
    seg_ids = jax.lax.broadcasted_iota(jnp.int32, (R, L), 1) & (Bn - 1)
    O = bb == seg_ids

    masked = jnp.where(O, gboth, -1e30)
    bmax = jnp.max(masked, axis=0, keepdims=True)
    m_old = m_ref[...]
    m_new = jnp.maximum(m_old, bmax)
    scale = jnp.exp(m_old - m_new)
    E = jnp.where(O, jnp.exp(gboth - m_new), 0.0)
    d_ref[...] = d_ref[...] * scale + jnp.sum(E, axis=0, keepdims=True)
    m_ref[...] = m_new

    f0 = jax.lax.dot_general(h[:, P * DH:(P + 1) * DH], fW2_ref[0],
                             (((1,), (0,)), ((), ())),
                             preferred_element_type=jnp.float32)
    f1 = jax.lax.dot_general(h[:, (P + 1) * DH:], fW2_ref[1],
                             (((1,), (0,)), ((), ())),
                             preferred_element_type=jnp.float32)
    Fcat = jnp.concatenate([f0, f1], axis=1).astype(jnp.bfloat16)
    Fcat = Fcat + fb2_ref[...]
    S_ref[...] = (S_ref[...] * jnp.transpose(scale)
                  + jax.lax.dot_general(E.astype(jnp.bfloat16), Fcat,
                                        (((0,), (0,)), ((), ())),
                                        preferred_element_type=jnp.float32))

    @pl.when(i == pl.num_programs(0) - 1)
    def _finish():
        for k in range(P):
            dT = jnp.transpose(d_ref[:, k * Bn:(k + 1) * Bn])
            Sk = S_ref[k * Bn:(k + 1) * Bn, k * DE:(k + 1) * DE]
            out_ref[k] = jnp.where(dT > 0.0, Sk / dT, 0.0)


def kernel(x, batch, n_nodes, Omegas, Phis, Lambdas, Omegas_norm, Phis_norm,
           Lambdas_norm, gate_W1, gate_b1, gate_W2, gate_b2, feat_W1, feat_b1,
           feat_W2, feat_b2):
    N, FD = x.shape
    Bn = n_nodes.shape[0]
    P, _, DH = gate_W1.shape
    DE = feat_W2.shape[2]
    R = 4000
    assert N % R == 0

    W1all = jnp.concatenate(
        [gate_W1[k] for k in range(P)] + [feat_W1[k] for k in range(P)],
        axis=1).astype(jnp.bfloat16)
    b1all = jnp.concatenate(
        [gate_b1[k] for k in range(P)]
        + [feat_b1[k] for k in range(P)])[None, :].astype(jnp.bfloat16)
    gW2w = jnp.zeros((P * DH, P * Bn), jnp.float32)
    gb2w = jnp.zeros((1, P * Bn), jnp.float32)
    for k in range(P):
        gW2w = gW2w.at[k * DH:(k + 1) * DH, k * Bn:(k + 1) * Bn].set(
            jnp.tile(gate_W2[k, :, 0:1], (1, Bn)))
        gb2w = gb2w.at[0, k * Bn:(k + 1) * Bn].set(gate_b2[k, 0])
    gW2w = gW2w.astype(jnp.bfloat16)
    fW2b = feat_W2.astype(jnp.bfloat16)
    fb2c = jnp.concatenate([feat_b2[k] for k in range(P)])[None, :].astype(
        jnp.bfloat16)
    batch2 = batch.astype(jnp.int32).reshape(N, 1)

    body = functools.partial(_body, R=R, P=P, Bn=Bn, DH=DH, DE=DE)
    pools = pl.pallas_call(
        body,
        grid=(N // R,),
        in_specs=[
            pl.BlockSpec((R, FD), lambda i: (i, 0)),
            pl.BlockSpec((R, 1), lambda i: (i, 0)),
            pl.BlockSpec((FD, 2 * P * DH), lambda i: (0, 0)),
            pl.BlockSpec((1, 2 * P * DH), lambda i: (0, 0)),
            pl.BlockSpec((P * DH, P * Bn), lambda i: (0, 0)),
            pl.BlockSpec((1, P * Bn), lambda i: (0, 0)),
            pl.BlockSpec((P, DH, DE), lambda i: (0, 0, 0)),
            pl.BlockSpec((1, P * DE), lambda i: (0, 0)),
        ],
        out_specs=pl.BlockSpec((P, Bn, DE), lambda i: (0, 0, 0)),
        out_shape=jax.ShapeDtypeStruct((P, Bn, DE), jnp.float32),
        scratch_shapes=[
            pltpu.VMEM((1, P * Bn), jnp.float32),
            pltpu.VMEM((1, P * Bn), jnp.float32),
            pltpu.VMEM((P * Bn, P * DE), jnp.float32),
        ],
    )(x, batch2, W1all, b1all, gW2w, gb2w, fW2b, fb2c)

    return jnp.concatenate(
        [pools[k] for k in range(P)]
        + [n_nodes, Omegas, Phis, Lambdas, Omegas_norm, Phis_norm,
           Lambdas_norm], axis=1)
